# baseline (speedup 1.0000x reference)
import functools

import jax
import jax.numpy as jnp
from jax import lax
from jax.experimental import pallas as pl
from jax.experimental.pallas import tpu as pltpu
from jax.experimental.pallas import tpu_sc as plsc

B = 16384
L = 200
D = 64
VOCAB = 1000000
H1 = 256
H2 = 128
C = 2
EPS = 1e-5

NC, NS = 2, 16
NW = NC * NS
CB = 4
LA = 104
LB = L - LA
NCHUNK = B // CB
CPW = NCHUNK // NW
NI2 = CPW // 2
NCG = D // 16


def _sc_pool(x_in, emb):
  mesh = plsc.VectorSubcoreMesh(core_axis_name="c", subcore_axis_name="s",
                                num_cores=NC, num_subcores=NS)

  @functools.partial(
      pl.kernel,
      out_type=jax.ShapeDtypeStruct((B, D), jnp.float32),
      mesh=mesh,
      compiler_params=pltpu.CompilerParams(use_tc_tiling_on_sc=False),
      scratch_types=[
          pltpu.VMEM((CB, L), jnp.int32),
          pltpu.VMEM((CB, L), jnp.int32),
          pltpu.VMEM((CB, L, D), jnp.float32),
          pltpu.VMEM((CB, L, D), jnp.float32),
          pltpu.VMEM((CB, D), jnp.float32),
          pltpu.SemaphoreType.DMA,
          pltpu.SemaphoreType.DMA,
      ],
  )
  def pool(x_hbm, emb_hbm, out_hbm, idx0, idx1, rows0, rows1, out_v, sem0,
           sem1):
    wid = lax.axis_index("s") * NC + lax.axis_index("c")
    base = wid * CPW

    def load_idx(chunk, idx_v):
      pltpu.sync_copy(x_hbm.at[pl.ds(chunk * CB, CB)], idx_v)

    def transfers(idx_v, rows_v, sem):
      for b in range(CB):
        yield (emb_hbm.at[idx_v.at[b, pl.ds(0, LA)]],
               rows_v.at[b, pl.ds(0, LA)], sem)
        yield (emb_hbm.at[idx_v.at[b, pl.ds(LA, LB)]],
               rows_v.at[b, pl.ds(LA, LB)], sem)

    def issue(idx_v, rows_v, sem):
      for src, dst, s in transfers(idx_v, rows_v, sem):
        pltpu.async_copy(src, dst, s)

    def drain(idx_v, rows_v, sem):
      for src, dst, s in transfers(idx_v, rows_v, sem):
        pltpu.make_async_copy(src, dst, s).wait()

    def reduce_store(rows_v, chunk):
      neg = jnp.full((16,), -jnp.inf, jnp.float32)

      def rbody(r, accs):
        out = []
        for b in range(CB):
          for c in range(NCG):
            v = rows_v[b, r, pl.ds(c * 16, 16)]
            out.append(jnp.maximum(accs[b * NCG + c], v))
        return tuple(out)

      accs = lax.fori_loop(0, L, rbody, (neg,) * (CB * NCG))
      for b in range(CB):
        for c in range(NCG):
          out_v[b, pl.ds(c * 16, 16)] = accs[b * NCG + c]
      pltpu.sync_copy(out_v, out_hbm.at[pl.ds(chunk * CB, CB)])

    load_idx(base, idx0)
    issue(idx0, rows0, sem0)

    def body2(i2, carry):
      a = base + 2 * i2
      load_idx(a + 1, idx1)
      issue(idx1, rows1, sem1)
      drain(idx0, rows0, sem0)
      reduce_store(rows0, a)

      @pl.when(i2 < NI2 - 1)
      def _():
        load_idx(a + 2, idx0)
        issue(idx0, rows0, sem0)

      drain(idx1, rows1, sem1)
      reduce_store(rows1, a + 1)
      return carry

    lax.fori_loop(0, NI2, body2, 0)

  return pool(x_in, emb)


def _tr_body(x_ref, o_ref):
  x = x_ref[...]
  eye = jnp.eye(D, dtype=jnp.float32)
  y = jax.lax.dot_general(x, eye, (((0,), (0,)), ((), ())),
                          preferred_element_type=jnp.float32)
  z = y.reshape(y.shape[0] // 2, 2, D)
  o_ref[...] = jnp.concatenate([z[:, 0, :], z[:, 1, :]], axis=1)


def _tc_pair_transpose(emb_t):
  TK = 8192
  grid = (pl.cdiv(VOCAB, TK),)
  return pl.pallas_call(
      _tr_body,
      grid=grid,
      in_specs=[pl.BlockSpec((D, TK), lambda i: (0, i))],
      out_specs=pl.BlockSpec((TK // 2, 2 * D), lambda i: (i, 0)),
      out_shape=jax.ShapeDtypeStruct((VOCAB // 2, 2 * D), jnp.float32),
  )(emb_t)


def _mlp_body(x_ref, w1_ref, b1_ref, w2_ref, b2_ref, w3_ref, b3_ref, o_ref):
  h = jnp.dot(x_ref[...], w1_ref[...],
              preferred_element_type=jnp.float32) + b1_ref[...]
  h = jnp.maximum(h, 0.0)
  h = jnp.dot(h, w2_ref[...], preferred_element_type=jnp.float32) + b2_ref[...]
  h = jnp.maximum(h, 0.0)
  o_ref[...] = jnp.dot(h, w3_ref[...],
                       preferred_element_type=jnp.float32) + b3_ref[...]


def _tc_mlp(pooled, W1, b1, W2f, b2f, W3p, b3p):
  MB = 2048
  return pl.pallas_call(
      _mlp_body,
      grid=(B // MB,),
      in_specs=[
          pl.BlockSpec((MB, D), lambda i: (i, 0)),
          pl.BlockSpec((D, H1), lambda i: (0, 0)),
          pl.BlockSpec((1, H1), lambda i: (0, 0)),
          pl.BlockSpec((H1, H2), lambda i: (0, 0)),
          pl.BlockSpec((1, H2), lambda i: (0, 0)),
          pl.BlockSpec((H2, 128), lambda i: (0, 0)),
          pl.BlockSpec((1, 128), lambda i: (0, 0)),
      ],
      out_specs=pl.BlockSpec((MB, 128), lambda i: (i, 0)),
      out_shape=jax.ShapeDtypeStruct((B, 128), jnp.float32),
  )(pooled, W1, b1.reshape(1, H1), W2f, b2f.reshape(1, H2), W3p,
    b3p.reshape(1, 128))


def kernel(x_in, emb, W1, b1, g1, be1, W2, b2, g2, be2, W3, b3):
  s = 1.0 / jnp.sqrt(jnp.float32(1.0 + EPS))
  W2f = (g1 * s)[:, None] * W2
  b2f = be1 @ W2 + b2
  W3f = (g2 * s)[:, None] * W3
  b3f = be2 @ W3 + b3
  W3p = jnp.zeros((H2, 128), jnp.float32).at[:, :C].set(W3f)
  b3p = jnp.zeros((128,), jnp.float32).at[:C].set(b3f)

  embR = emb
  pooled = _sc_pool(x_in, embR)
  logits = _tc_mlp(pooled, W1, b1, W2f, b2f, W3p, b3p)
  return logits[:, :C]


# speedup vs baseline: 1.1873x; 1.1873x over previous
import functools

import jax
import jax.numpy as jnp
from jax import lax
from jax.experimental import pallas as pl
from jax.experimental.pallas import tpu as pltpu
from jax.experimental.pallas import tpu_sc as plsc

B = 16384
L = 200
D = 64
VOCAB = 1000000
H1 = 256
H2 = 128
C = 2
EPS = 1e-5

NC, NS = 2, 16
NW = NC * NS
CB = 4
LA = 104
LB = L - LA
NCHUNK = B // CB
CPW = NCHUNK // NW
NI2 = CPW // 2
NCG = D // 16


def _sc_pool(x_in, emb):
  mesh = plsc.VectorSubcoreMesh(core_axis_name="c", subcore_axis_name="s",
                                num_cores=NC, num_subcores=NS)

  @functools.partial(
      pl.kernel,
      out_type=jax.ShapeDtypeStruct((B, D), jnp.float32),
      mesh=mesh,
      compiler_params=pltpu.CompilerParams(use_tc_tiling_on_sc=False),
      scratch_types=[
          pltpu.VMEM((CB, L), jnp.int32),
          pltpu.VMEM((CB, L), jnp.int32),
          pltpu.VMEM((CB, L, D), jnp.float32),
          pltpu.VMEM((CB, L, D), jnp.float32),
          pltpu.VMEM((CB, D), jnp.float32),
          pltpu.SemaphoreType.DMA,
          pltpu.SemaphoreType.DMA,
      ],
  )
  def pool(x_hbm, emb_hbm, out_hbm, idx0, idx1, rows0, rows1, out_v, sem0,
           sem1):
    wid = lax.axis_index("s") * NC + lax.axis_index("c")
    base = wid * CPW

    def load_idx(chunk, idx_v):
      pltpu.sync_copy(x_hbm.at[pl.ds(chunk * CB, CB)], idx_v)

    def transfers(idx_v, rows_v, sem):
      for b in range(CB):
        yield (emb_hbm.at[idx_v.at[b, pl.ds(0, LA)]],
               rows_v.at[b, pl.ds(0, LA)], sem)
        yield (emb_hbm.at[idx_v.at[b, pl.ds(LA, LB)]],
               rows_v.at[b, pl.ds(LA, LB)], sem)

    def issue(idx_v, rows_v, sem):
      for src, dst, s in transfers(idx_v, rows_v, sem):
        pltpu.async_copy(src, dst, s)

    def drain(idx_v, rows_v, sem):
      for src, dst, s in transfers(idx_v, rows_v, sem):
        pltpu.make_async_copy(src, dst, s).wait()

    def reduce_store(rows_v, chunk):
      neg = jnp.full((16,), -jnp.inf, jnp.float32)

      def rbody(r, accs):
        out = []
        for b in range(CB):
          for c in range(NCG):
            v = rows_v[b, r, pl.ds(c * 16, 16)]
            out.append(jnp.maximum(accs[b * NCG + c], v))
        return tuple(out)

      accs = lax.fori_loop(0, L, rbody, (neg,) * (CB * NCG))
      for b in range(CB):
        for c in range(NCG):
          out_v[b, pl.ds(c * 16, 16)] = accs[b * NCG + c]
      pltpu.sync_copy(out_v, out_hbm.at[pl.ds(chunk * CB, CB)])

    load_idx(base, idx0)
    issue(idx0, rows0, sem0)

    def body2(i2, carry):
      a = base + 2 * i2
      load_idx(a + 1, idx1)
      issue(idx1, rows1, sem1)
      drain(idx0, rows0, sem0)
      reduce_store(rows0, a)

      @pl.when(i2 < NI2 - 1)
      def _():
        load_idx(a + 2, idx0)
        issue(idx0, rows0, sem0)

      drain(idx1, rows1, sem1)
      reduce_store(rows1, a + 1)
      return carry

    lax.fori_loop(0, NI2, body2, 0)

  return pool(x_in, emb)


def _tr_body(x_ref, o_ref):
  x = x_ref[...]
  eye = jnp.eye(D, dtype=jnp.float32)
  y = jax.lax.dot_general(x, eye, (((0,), (0,)), ((), ())),
                          preferred_element_type=jnp.float32)
  z = y.reshape(y.shape[0] // 2, 2, D)
  o_ref[...] = jnp.concatenate([z[:, 0, :], z[:, 1, :]], axis=1)


def _tc_pair_transpose(emb_t):
  TK = 8192
  grid = (pl.cdiv(VOCAB, TK),)
  return pl.pallas_call(
      _tr_body,
      grid=grid,
      in_specs=[pl.BlockSpec((D, TK), lambda i: (0, i))],
      out_specs=pl.BlockSpec((TK // 2, 2 * D), lambda i: (i, 0)),
      out_shape=jax.ShapeDtypeStruct((VOCAB // 2, 2 * D), jnp.float32),
      compiler_params=pltpu.CompilerParams(
          dimension_semantics=("parallel",)),
  )(emb_t)


def _mlp_body(x_ref, w1_ref, b1_ref, w2_ref, b2_ref, w3_ref, b3_ref, o_ref):
  h = jnp.dot(x_ref[...], w1_ref[...],
              preferred_element_type=jnp.float32) + b1_ref[...]
  h = jnp.maximum(h, 0.0)
  h = jnp.dot(h, w2_ref[...], preferred_element_type=jnp.float32) + b2_ref[...]
  h = jnp.maximum(h, 0.0)
  o_ref[...] = jnp.dot(h, w3_ref[...],
                       preferred_element_type=jnp.float32) + b3_ref[...]


def _tc_mlp(pooled, W1, b1, W2f, b2f, W3p, b3p):
  MB = 2048
  return pl.pallas_call(
      _mlp_body,
      grid=(B // MB,),
      in_specs=[
          pl.BlockSpec((MB, D), lambda i: (i, 0)),
          pl.BlockSpec((D, H1), lambda i: (0, 0)),
          pl.BlockSpec((1, H1), lambda i: (0, 0)),
          pl.BlockSpec((H1, H2), lambda i: (0, 0)),
          pl.BlockSpec((1, H2), lambda i: (0, 0)),
          pl.BlockSpec((H2, 128), lambda i: (0, 0)),
          pl.BlockSpec((1, 128), lambda i: (0, 0)),
      ],
      out_specs=pl.BlockSpec((MB, 128), lambda i: (i, 0)),
      out_shape=jax.ShapeDtypeStruct((B, 128), jnp.float32),
  )(pooled, W1, b1.reshape(1, H1), W2f, b2f.reshape(1, H2), W3p,
    b3p.reshape(1, 128))


def kernel(x_in, emb, W1, b1, g1, be1, W2, b2, g2, be2, W3, b3):
  s = 1.0 / jnp.sqrt(jnp.float32(1.0 + EPS))
  W2f = (g1 * s)[:, None] * W2
  b2f = be1 @ W2 + b2
  W3f = (g2 * s)[:, None] * W3
  b3f = be2 @ W3 + b3
  W3p = jnp.zeros((H2, 128), jnp.float32).at[:, :C].set(W3f)
  b3p = jnp.zeros((128,), jnp.float32).at[:C].set(b3f)

  embR = _tc_pair_transpose(emb.T).reshape(VOCAB, D)
  pooled = _sc_pool(x_in, embR)
  logits = _tc_mlp(pooled, W1, b1, W2f, b2f, W3p, b3p)
  return logits[:, :C]


# speedup vs baseline: 1.5908x; 1.3398x over previous
import functools

import jax
import jax.numpy as jnp
from jax import lax
from jax.experimental import pallas as pl
from jax.experimental.pallas import tpu as pltpu
from jax.experimental.pallas import tpu_sc as plsc

B = 16384
L = 200
D = 64
VOCAB = 1000000
H1 = 256
H2 = 128
C = 2
EPS = 1e-5

NC, NS = 2, 16
NW = NC * NS
CB = 4
LA = 104
LB = L - LA
NCHUNK = B // CB
CPW = NCHUNK // NW
NI2 = CPW // 2
NCG = D // 16


def _sc_pool(x_in, emb):
  mesh = plsc.VectorSubcoreMesh(core_axis_name="c", subcore_axis_name="s",
                                num_cores=NC, num_subcores=NS)

  @functools.partial(
      pl.kernel,
      out_type=jax.ShapeDtypeStruct((B, D), jnp.float32),
      mesh=mesh,
      compiler_params=pltpu.CompilerParams(use_tc_tiling_on_sc=False),
      scratch_types=[
          pltpu.VMEM((CB, L), jnp.int32),
          pltpu.VMEM((CB, L), jnp.int32),
          pltpu.VMEM((CB, L, D), jnp.float32),
          pltpu.VMEM((CB, L, D), jnp.float32),
          pltpu.VMEM((CB, D), jnp.float32),
          pltpu.SemaphoreType.DMA,
          pltpu.SemaphoreType.DMA,
      ],
  )
  def pool(x_hbm, emb_hbm, out_hbm, idx0, idx1, rows0, rows1, out_v, sem0,
           sem1):
    wid = lax.axis_index("s") * NC + lax.axis_index("c")
    base = wid * CPW

    def load_idx(chunk, idx_v):
      pltpu.sync_copy(x_hbm.at[pl.ds(chunk * CB, CB)], idx_v)

    def transfers(idx_v, rows_v, sem):
      for b in range(CB):
        yield (emb_hbm.at[idx_v.at[b, pl.ds(0, LA)]],
               rows_v.at[b, pl.ds(0, LA)], sem)
        yield (emb_hbm.at[idx_v.at[b, pl.ds(LA, LB)]],
               rows_v.at[b, pl.ds(LA, LB)], sem)

    def issue(idx_v, rows_v, sem):
      for src, dst, s in transfers(idx_v, rows_v, sem):
        pltpu.async_copy(src, dst, s)

    def drain(idx_v, rows_v, sem):
      for src, dst, s in transfers(idx_v, rows_v, sem):
        pltpu.make_async_copy(src, dst, s).wait()

    def reduce_store(rows_v, chunk):
      neg = jnp.full((16,), -jnp.inf, jnp.float32)

      def rbody(r, accs):
        out = []
        for b in range(CB):
          for c in range(NCG):
            v = rows_v[b, r, pl.ds(c * 16, 16)]
            out.append(jnp.maximum(accs[b * NCG + c], v))
        return tuple(out)

      accs = lax.fori_loop(0, L, rbody, (neg,) * (CB * NCG))
      for b in range(CB):
        for c in range(NCG):
          out_v[b, pl.ds(c * 16, 16)] = accs[b * NCG + c]
      pltpu.sync_copy(out_v, out_hbm.at[pl.ds(chunk * CB, CB)])

    load_idx(base, idx0)
    issue(idx0, rows0, sem0)

    def body2(i2, carry):
      a = base + 2 * i2
      load_idx(a + 1, idx1)
      issue(idx1, rows1, sem1)
      drain(idx0, rows0, sem0)
      reduce_store(rows0, a)

      @pl.when(i2 < NI2 - 1)
      def _():
        load_idx(a + 2, idx0)
        issue(idx0, rows0, sem0)

      drain(idx1, rows1, sem1)
      reduce_store(rows1, a + 1)
      return carry

    lax.fori_loop(0, NI2, body2, 0)

  return pool(x_in, emb)


TK = 8192
TKH = TK // 2
NBLK = pl.cdiv(VOCAB, TK)
VROWS = NBLK * TKH


def _tr_body(x_ref, o_ref):
  x = x_ref[...]
  xc = jnp.concatenate([x[:, :TKH], x[:, TKH:]], axis=0)
  eye = jnp.eye(2 * D, dtype=jnp.float32)
  o_ref[...] = jax.lax.dot_general(xc, eye, (((0,), (0,)), ((), ())),
                                   preferred_element_type=jnp.float32)


def _tc_halves_transpose(emb_t):
  return pl.pallas_call(
      _tr_body,
      grid=(NBLK,),
      in_specs=[pl.BlockSpec((D, TK), lambda i: (0, i))],
      out_specs=pl.BlockSpec((TKH, 2 * D), lambda i: (i, 0)),
      out_shape=jax.ShapeDtypeStruct((VROWS, 2 * D), jnp.float32),
  )(emb_t)


def _ix_body(x_ref, o_ref):
  i = x_ref[...]
  o_ref[...] = ((i >> 13) << 13) | ((i & 4095) << 1) | ((i >> 12) & 1)


def _tc_idx_transform(x_in):
  MB = 2048
  return pl.pallas_call(
      _ix_body,
      grid=(B // MB,),
      in_specs=[pl.BlockSpec((MB, L), lambda i: (i, 0))],
      out_specs=pl.BlockSpec((MB, L), lambda i: (i, 0)),
      out_shape=jax.ShapeDtypeStruct((B, L), jnp.int32),
  )(x_in)


def _mlp_body(x_ref, w1_ref, b1_ref, w2_ref, b2_ref, w3_ref, b3_ref, o_ref):
  h = jnp.dot(x_ref[...], w1_ref[...],
              preferred_element_type=jnp.float32) + b1_ref[...]
  h = jnp.maximum(h, 0.0)
  h = jnp.dot(h, w2_ref[...], preferred_element_type=jnp.float32) + b2_ref[...]
  h = jnp.maximum(h, 0.0)
  o_ref[...] = jnp.dot(h, w3_ref[...],
                       preferred_element_type=jnp.float32) + b3_ref[...]


def _tc_mlp(pooled, W1, b1, W2f, b2f, W3p, b3p):
  MB = 2048
  return pl.pallas_call(
      _mlp_body,
      grid=(B // MB,),
      in_specs=[
          pl.BlockSpec((MB, D), lambda i: (i, 0)),
          pl.BlockSpec((D, H1), lambda i: (0, 0)),
          pl.BlockSpec((1, H1), lambda i: (0, 0)),
          pl.BlockSpec((H1, H2), lambda i: (0, 0)),
          pl.BlockSpec((1, H2), lambda i: (0, 0)),
          pl.BlockSpec((H2, 128), lambda i: (0, 0)),
          pl.BlockSpec((1, 128), lambda i: (0, 0)),
      ],
      out_specs=pl.BlockSpec((MB, 128), lambda i: (i, 0)),
      out_shape=jax.ShapeDtypeStruct((B, 128), jnp.float32),
  )(pooled, W1, b1.reshape(1, H1), W2f, b2f.reshape(1, H2), W3p,
    b3p.reshape(1, 128))


def kernel(x_in, emb, W1, b1, g1, be1, W2, b2, g2, be2, W3, b3):
  s = 1.0 / jnp.sqrt(jnp.float32(1.0 + EPS))
  W2f = (g1 * s)[:, None] * W2
  b2f = be1 @ W2 + b2
  W3f = (g2 * s)[:, None] * W3
  b3f = be2 @ W3 + b3
  W3p = jnp.zeros((H2, 128), jnp.float32).at[:, :C].set(W3f)
  b3p = jnp.zeros((128,), jnp.float32).at[:C].set(b3f)

  embR = _tc_halves_transpose(emb.T).reshape(2 * VROWS, D)
  pooled = _sc_pool(_tc_idx_transform(x_in), embR)
  logits = _tc_mlp(pooled, W1, b1, W2f, b2f, W3p, b3p)
  return logits[:, :C]


# speedup vs baseline: 1.6290x; 1.0240x over previous
import functools

import jax
import jax.numpy as jnp
from jax import lax
from jax.experimental import pallas as pl
from jax.experimental.pallas import tpu as pltpu
from jax.experimental.pallas import tpu_sc as plsc

B = 16384
L = 200
D = 64
VOCAB = 1000000
H1 = 256
H2 = 128
C = 2
EPS = 1e-5

NC, NS = 2, 16
NW = NC * NS
CB = 4
LA = 104
LB = L - LA
NCHUNK = B // CB
CPW = NCHUNK // NW
NI2 = CPW // 2
NCG = D // 16


def _sc_pool(x_in, emb):
  mesh = plsc.VectorSubcoreMesh(core_axis_name="c", subcore_axis_name="s",
                                num_cores=NC, num_subcores=NS)

  @functools.partial(
      pl.kernel,
      out_type=jax.ShapeDtypeStruct((B, D), jnp.float32),
      mesh=mesh,
      compiler_params=pltpu.CompilerParams(use_tc_tiling_on_sc=False),
      scratch_types=[
          pltpu.VMEM((CB, L), jnp.int32),
          pltpu.VMEM((CB, L), jnp.int32),
          pltpu.VMEM((CB, L, D), jnp.float32),
          pltpu.VMEM((CB, L, D), jnp.float32),
          pltpu.VMEM((CB, D), jnp.float32),
          pltpu.SemaphoreType.DMA,
          pltpu.SemaphoreType.DMA,
      ],
  )
  def pool(x_hbm, emb_hbm, out_hbm, idx0, idx1, rows0, rows1, out_v, sem0,
           sem1):
    wid = lax.axis_index("s") * NC + lax.axis_index("c")
    base = wid * CPW

    def load_idx(chunk, idx_v):
      pltpu.sync_copy(x_hbm.at[pl.ds(chunk * CB, CB)], idx_v)

    def transfers(idx_v, rows_v, sem):
      for b in range(CB):
        yield (emb_hbm.at[idx_v.at[b, pl.ds(0, LA)]],
               rows_v.at[b, pl.ds(0, LA)], sem)
        yield (emb_hbm.at[idx_v.at[b, pl.ds(LA, LB)]],
               rows_v.at[b, pl.ds(LA, LB)], sem)

    def issue(idx_v, rows_v, sem):
      for src, dst, s in transfers(idx_v, rows_v, sem):
        pltpu.async_copy(src, dst, s)

    def drain(idx_v, rows_v, sem):
      for src, dst, s in transfers(idx_v, rows_v, sem):
        pltpu.make_async_copy(src, dst, s).wait()

    def reduce_store(rows_v, chunk):
      neg = jnp.full((16,), -jnp.inf, jnp.float32)

      def rbody(r, accs):
        out = []
        for b in range(CB):
          for c in range(NCG):
            v = rows_v[b, r, pl.ds(c * 16, 16)]
            out.append(jnp.maximum(accs[b * NCG + c], v))
        return tuple(out)

      accs = lax.fori_loop(0, L, rbody, (neg,) * (CB * NCG))
      for b in range(CB):
        for c in range(NCG):
          out_v[b, pl.ds(c * 16, 16)] = accs[b * NCG + c]
      pltpu.sync_copy(out_v, out_hbm.at[pl.ds(chunk * CB, CB)])

    load_idx(base, idx0)
    issue(idx0, rows0, sem0)

    def body2(i2, carry):
      a = base + 2 * i2
      load_idx(a + 1, idx1)
      issue(idx1, rows1, sem1)
      drain(idx0, rows0, sem0)
      reduce_store(rows0, a)

      @pl.when(i2 < NI2 - 1)
      def _():
        load_idx(a + 2, idx0)
        issue(idx0, rows0, sem0)

      drain(idx1, rows1, sem1)
      reduce_store(rows1, a + 1)
      return carry

    lax.fori_loop(0, NI2, body2, 0)

  return pool(x_in, emb)


TK = 16384
TKH = TK // 2
NBLK = pl.cdiv(VOCAB, TK)
VROWS = NBLK * TKH


def _tr_body(x_ref, o_ref):
  x = x_ref[...]
  xc = jnp.concatenate([x[:, :TKH], x[:, TKH:]], axis=0)
  eye = jnp.eye(2 * D, dtype=jnp.float32)
  o_ref[...] = jax.lax.dot_general(xc, eye, (((0,), (0,)), ((), ())),
                                   preferred_element_type=jnp.float32)


def _tc_halves_transpose(emb_t):
  return pl.pallas_call(
      _tr_body,
      grid=(NBLK,),
      in_specs=[pl.BlockSpec((D, TK), lambda i: (0, i))],
      out_specs=pl.BlockSpec((TKH, 2 * D), lambda i: (i, 0)),
      out_shape=jax.ShapeDtypeStruct((VROWS, 2 * D), jnp.float32),
  )(emb_t)


def _ix_body(x_ref, o_ref):
  i = x_ref[...]
  a = (i // TK) * TK
  t = i % TK
  o_ref[...] = a | ((t % TKH) << 1) | (t // TKH)


def _tc_idx_transform(x_in):
  MB = 2048
  return pl.pallas_call(
      _ix_body,
      grid=(B // MB,),
      in_specs=[pl.BlockSpec((MB, L), lambda i: (i, 0))],
      out_specs=pl.BlockSpec((MB, L), lambda i: (i, 0)),
      out_shape=jax.ShapeDtypeStruct((B, L), jnp.int32),
  )(x_in)


def _mlp_body(x_ref, w1_ref, b1_ref, w2_ref, b2_ref, w3_ref, b3_ref, o_ref):
  h = jnp.dot(x_ref[...], w1_ref[...],
              preferred_element_type=jnp.float32) + b1_ref[...]
  h = jnp.maximum(h, 0.0)
  h = jnp.dot(h, w2_ref[...], preferred_element_type=jnp.float32) + b2_ref[...]
  h = jnp.maximum(h, 0.0)
  o_ref[...] = jnp.dot(h, w3_ref[...],
                       preferred_element_type=jnp.float32) + b3_ref[...]


def _tc_mlp(pooled, W1, b1, W2f, b2f, W3p, b3p):
  MB = 2048
  return pl.pallas_call(
      _mlp_body,
      grid=(B // MB,),
      in_specs=[
          pl.BlockSpec((MB, D), lambda i: (i, 0)),
          pl.BlockSpec((D, H1), lambda i: (0, 0)),
          pl.BlockSpec((1, H1), lambda i: (0, 0)),
          pl.BlockSpec((H1, H2), lambda i: (0, 0)),
          pl.BlockSpec((1, H2), lambda i: (0, 0)),
          pl.BlockSpec((H2, 128), lambda i: (0, 0)),
          pl.BlockSpec((1, 128), lambda i: (0, 0)),
      ],
      out_specs=pl.BlockSpec((MB, 128), lambda i: (i, 0)),
      out_shape=jax.ShapeDtypeStruct((B, 128), jnp.float32),
  )(pooled, W1, b1.reshape(1, H1), W2f, b2f.reshape(1, H2), W3p,
    b3p.reshape(1, 128))


def kernel(x_in, emb, W1, b1, g1, be1, W2, b2, g2, be2, W3, b3):
  s = 1.0 / jnp.sqrt(jnp.float32(1.0 + EPS))
  W2f = (g1 * s)[:, None] * W2
  b2f = be1 @ W2 + b2
  W3f = (g2 * s)[:, None] * W3
  b3f = be2 @ W3 + b3
  W3p = jnp.zeros((H2, 128), jnp.float32).at[:, :C].set(W3f)
  b3p = jnp.zeros((128,), jnp.float32).at[:C].set(b3f)

  embR = _tc_halves_transpose(emb.T).reshape(2 * VROWS, D)
  pooled = _sc_pool(_tc_idx_transform(x_in), embR)
  logits = _tc_mlp(pooled, W1, b1, W2f, b2f, W3p, b3p)
  return logits[:, :C]


# speedup vs baseline: 1.6385x; 1.0059x over previous
import functools

import jax
import jax.numpy as jnp
from jax import lax
from jax.experimental import pallas as pl
from jax.experimental.pallas import tpu as pltpu
from jax.experimental.pallas import tpu_sc as plsc

B = 16384
L = 200
D = 64
VOCAB = 1000000
H1 = 256
H2 = 128
C = 2
EPS = 1e-5

NC, NS = 2, 16
NW = NC * NS
CB = 4
LA = 104
LB = L - LA
NCHUNK = B // CB
CPW = NCHUNK // NW
NI2 = CPW // 2
NCG = D // 16


def _sc_pool(x_in, emb):
  mesh = plsc.VectorSubcoreMesh(core_axis_name="c", subcore_axis_name="s",
                                num_cores=NC, num_subcores=NS)

  @functools.partial(
      pl.kernel,
      out_type=jax.ShapeDtypeStruct((B, D), jnp.float32),
      mesh=mesh,
      compiler_params=pltpu.CompilerParams(use_tc_tiling_on_sc=False),
      scratch_types=[
          pltpu.VMEM((CB, L), jnp.int32),
          pltpu.VMEM((CB, L), jnp.int32),
          pltpu.VMEM((CB, L, D), jnp.float32),
          pltpu.VMEM((CB, L, D), jnp.float32),
          pltpu.VMEM((CB, D), jnp.float32),
          pltpu.SemaphoreType.DMA,
          pltpu.SemaphoreType.DMA,
      ],
  )
  def pool(x_hbm, emb_hbm, out_hbm, idx0, idx1, rows0, rows1, out_v, sem0,
           sem1):
    wid = lax.axis_index("s") * NC + lax.axis_index("c")
    base = wid * CPW

    def load_idx(chunk, idx_v):
      pltpu.sync_copy(x_hbm.at[pl.ds(chunk * CB, CB)], idx_v)

    def transfers(idx_v, rows_v, sem):
      for b in range(CB):
        yield (emb_hbm.at[idx_v.at[b, pl.ds(0, LA)]],
               rows_v.at[b, pl.ds(0, LA)], sem)
        yield (emb_hbm.at[idx_v.at[b, pl.ds(LA, LB)]],
               rows_v.at[b, pl.ds(LA, LB)], sem)

    def issue(idx_v, rows_v, sem):
      for src, dst, s in transfers(idx_v, rows_v, sem):
        pltpu.async_copy(src, dst, s)

    def drain(idx_v, rows_v, sem):
      for src, dst, s in transfers(idx_v, rows_v, sem):
        pltpu.make_async_copy(src, dst, s).wait()

    def reduce_store(rows_v, chunk):
      neg = jnp.full((16,), -jnp.inf, jnp.float32)

      def rbody(r, accs):
        out = []
        for b in range(CB):
          for c in range(NCG):
            v = rows_v[b, r, pl.ds(c * 16, 16)]
            out.append(jnp.maximum(accs[b * NCG + c], v))
        return tuple(out)

      accs = lax.fori_loop(0, L, rbody, (neg,) * (CB * NCG))
      for b in range(CB):
        for c in range(NCG):
          out_v[b, pl.ds(c * 16, 16)] = accs[b * NCG + c]
      pltpu.sync_copy(out_v, out_hbm.at[pl.ds(chunk * CB, CB)])

    load_idx(base, idx0)
    issue(idx0, rows0, sem0)

    def body2(i2, carry):
      a = base + 2 * i2
      load_idx(a + 1, idx1)
      issue(idx1, rows1, sem1)
      drain(idx0, rows0, sem0)
      reduce_store(rows0, a)

      @pl.when(i2 < NI2 - 1)
      def _():
        load_idx(a + 2, idx0)
        issue(idx0, rows0, sem0)

      drain(idx1, rows1, sem1)
      reduce_store(rows1, a + 1)
      return carry

    lax.fori_loop(0, NI2, body2, 0)

  return pool(x_in, emb)


TK = 32768
TKH = TK // 2
NBLK = pl.cdiv(VOCAB, TK)
VROWS = NBLK * TKH


def _tr_body(x_ref, o_ref):
  x = x_ref[...]
  xc = jnp.concatenate([x[:, :TKH], x[:, TKH:]], axis=0)
  eye = jnp.eye(2 * D, dtype=jnp.float32)
  o_ref[...] = jax.lax.dot_general(xc, eye, (((0,), (0,)), ((), ())),
                                   preferred_element_type=jnp.float32)


def _tc_halves_transpose(emb_t):
  return pl.pallas_call(
      _tr_body,
      grid=(NBLK,),
      in_specs=[pl.BlockSpec((D, TK), lambda i: (0, i))],
      out_specs=pl.BlockSpec((TKH, 2 * D), lambda i: (i, 0)),
      out_shape=jax.ShapeDtypeStruct((VROWS, 2 * D), jnp.float32),
  )(emb_t)


def _ix_body(x_ref, o_ref):
  i = x_ref[...]
  a = (i // TK) * TK
  t = i % TK
  o_ref[...] = a | ((t % TKH) << 1) | (t // TKH)


def _tc_idx_transform(x_in):
  MB = 2048
  return pl.pallas_call(
      _ix_body,
      grid=(B // MB,),
      in_specs=[pl.BlockSpec((MB, L), lambda i: (i, 0))],
      out_specs=pl.BlockSpec((MB, L), lambda i: (i, 0)),
      out_shape=jax.ShapeDtypeStruct((B, L), jnp.int32),
  )(x_in)


def _mlp_body(x_ref, w1_ref, b1_ref, w2_ref, b2_ref, w3_ref, b3_ref, o_ref):
  h = jnp.dot(x_ref[...], w1_ref[...],
              preferred_element_type=jnp.float32) + b1_ref[...]
  h = jnp.maximum(h, 0.0)
  h = jnp.dot(h, w2_ref[...], preferred_element_type=jnp.float32) + b2_ref[...]
  h = jnp.maximum(h, 0.0)
  o_ref[...] = jnp.dot(h, w3_ref[...],
                       preferred_element_type=jnp.float32) + b3_ref[...]


def _tc_mlp(pooled, W1, b1, W2f, b2f, W3p, b3p):
  MB = 2048
  return pl.pallas_call(
      _mlp_body,
      grid=(B // MB,),
      in_specs=[
          pl.BlockSpec((MB, D), lambda i: (i, 0)),
          pl.BlockSpec((D, H1), lambda i: (0, 0)),
          pl.BlockSpec((1, H1), lambda i: (0, 0)),
          pl.BlockSpec((H1, H2), lambda i: (0, 0)),
          pl.BlockSpec((1, H2), lambda i: (0, 0)),
          pl.BlockSpec((H2, 128), lambda i: (0, 0)),
          pl.BlockSpec((1, 128), lambda i: (0, 0)),
      ],
      out_specs=pl.BlockSpec((MB, 128), lambda i: (i, 0)),
      out_shape=jax.ShapeDtypeStruct((B, 128), jnp.float32),
  )(pooled, W1, b1.reshape(1, H1), W2f, b2f.reshape(1, H2), W3p,
    b3p.reshape(1, 128))


def kernel(x_in, emb, W1, b1, g1, be1, W2, b2, g2, be2, W3, b3):
  s = 1.0 / jnp.sqrt(jnp.float32(1.0 + EPS))
  W2f = (g1 * s)[:, None] * W2
  b2f = be1 @ W2 + b2
  W3f = (g2 * s)[:, None] * W3
  b3f = be2 @ W3 + b3
  W3p = jnp.zeros((H2, 128), jnp.float32).at[:, :C].set(W3f)
  b3p = jnp.zeros((128,), jnp.float32).at[:C].set(b3f)

  embR = _tc_halves_transpose(emb.T).reshape(2 * VROWS, D)
  pooled = _sc_pool(_tc_idx_transform(x_in), embR)
  logits = _tc_mlp(pooled, W1, b1, W2f, b2f, W3p, b3p)
  return logits[:, :C]
